# baseline (speedup 1.0000x reference)
import functools

import jax
import jax.numpy as jnp
import numpy as np
from jax import lax
from jax.experimental import pallas as pl
from jax.experimental.pallas import tpu as pltpu
from jax.experimental.pallas import tpu_sc as plsc

NC = 2
NS = 16
NW = NC * NS

D = 64
PERIOD = 200
CHUNK = 100
VB = 512
NBLK = 1954


def _pos_table(seq_len, d_model):
    pos = jnp.arange(seq_len, dtype=jnp.float32)[:, None]
    div = jnp.exp(jnp.arange(0, d_model, 2, dtype=jnp.float32)
                  * -(np.log(10000.0) / d_model))
    pe = jnp.zeros((seq_len, d_model), dtype=jnp.float32)
    pe = pe.at[:, 0::2].set(jnp.sin(pos * div))
    pe = pe.at[:, 1::2].set(jnp.cos(pos * div))
    return pe


def _phase_a(tabT):
    def _bf16_bits(x):
        u = jax.lax.bitcast_convert_type(x, jnp.int32)
        lsb = lax.shift_right_logical(u, 16) & 1
        return lax.shift_right_logical(u + 0x7FFF + lsb, 16)

    def body(in_ref, out_ref):
        t = in_ref[...].T
        lo = _bf16_bits(t[:, : D // 2])
        hi = _bf16_bits(t[:, D // 2:])
        w = lax.shift_left(hi, 16) | lo
        q = VB // 4
        out_ref[...] = jnp.concatenate(
            [w[:q], w[q:2 * q], w[2 * q:3 * q], w[3 * q:]], axis=1)

    return pl.pallas_call(
        body,
        grid=(NBLK,),
        in_specs=[pl.BlockSpec((D, VB), lambda i: (0, i))],
        out_specs=pl.BlockSpec((VB // 4, 128), lambda i: (i, 0)),
        out_shape=jax.ShapeDtypeStruct((NBLK * VB // 4, 128), jnp.int32),
    )(tabT)


def _phase_b(n_rows, n_flat):
    per_w = n_rows // NW
    n_chunks = per_w // CHUNK
    mesh = plsc.VectorSubcoreMesh(core_axis_name="c", subcore_axis_name="s")

    @functools.partial(
        pl.kernel,
        mesh=mesh,
        compiler_params=pltpu.CompilerParams(
            use_tc_tiling_on_sc=False, needs_layout_passes=False),
        out_type=jax.ShapeDtypeStruct((n_rows, 128), jnp.float32),
        scratch_types=[
            pltpu.VMEM((n_chunks, CHUNK), jnp.int32),
            pltpu.VMEM((PERIOD, D), jnp.float32),
            pltpu.VMEM((4, CHUNK, D // 2), jnp.int32),
            pltpu.VMEM((4, CHUNK, D), jnp.float32),
            pltpu.SemaphoreType.DMA((4,)),
            pltpu.SemaphoreType.DMA((4,)),
        ],
    )
    def body(tab_hbm, idx_hbm, pe_hbm, out_hbm, idx_v, pe_v, gbuf, sbuf,
             gsem, ssem):
        wid = lax.axis_index("s") * NC + lax.axis_index("c")
        rbase = wid * per_w

        pltpu.sync_copy(idx_hbm.at[wid], idx_v)
        pltpu.sync_copy(pe_hbm, pe_v)

        def fire_gather(j, b):
            pltpu.async_copy(tab_hbm.at[idx_v.at[j]], gbuf.at[b], gsem.at[b])

        def wait_gather(j, b):
            pltpu.make_async_copy(
                tab_hbm.at[idx_v.at[j]], gbuf.at[b], gsem.at[b]).wait()

        def fire_store(j, b):
            pltpu.async_copy(sbuf.at[b],
                             out_hbm.at[pl.ds(rbase + j * CHUNK, CHUNK),
                                        pl.ds(0, D)],
                             ssem.at[b])

        def wait_store(j, b):
            pltpu.make_async_copy(
                sbuf.at[b],
                out_hbm.at[pl.ds(rbase + j * CHUNK, CHUNK), pl.ds(0, D)],
                ssem.at[b]).wait()

        def add_pe(b, parity):
            pbase = parity * CHUNK

            def row_body(r, carry):
                for g in range(D // 32):
                    x32 = gbuf[b, r, pl.ds(g * 16, 16)]
                    x = plsc.bitcast(x32, jnp.bfloat16)
                    lo, hi = plsc.unpack(x, format=plsc.PackFormat.INTERLEAVED)
                    c0, c1 = g * 16, D // 2 + g * 16
                    sbuf[b, r, pl.ds(c0, 16)] = lo + pe_v[pbase + r, pl.ds(c0, 16)]
                    sbuf[b, r, pl.ds(c1, 16)] = hi + pe_v[pbase + r, pl.ds(c1, 16)]
                return carry

            lax.fori_loop(0, CHUNK, row_body, 0, unroll=2)

        fire_gather(0, 0)
        fire_gather(1, 1)

        wait_gather(0, 0)
        add_pe(0, 0)
        fire_store(0, 0)
        fire_gather(2, 2)

        wait_gather(1, 1)
        add_pe(1, 1)
        fire_store(1, 1)
        fire_gather(3, 3)

        def steady(jj, carry):
            j0 = 2 + jj * 4
            for b_off in range(4):
                j = j0 + b_off
                b = (2 + b_off) % 4
                parity = b_off % 2
                wait_gather(j, b)
                wait_store(j - 2, (b + 2) % 4)
                add_pe(b, parity)
                fire_store(j, b)
                fire_gather(j + 2, (b + 2) % 4)
            return carry

        lax.fori_loop(0, (n_chunks - 4) // 4, steady, 0)

        jt = n_chunks - 2
        wait_gather(jt, jt % 4)
        wait_store(jt - 2, (jt - 2) % 4)
        add_pe(jt % 4, jt % 2)
        fire_store(jt, jt % 4)

        jt = n_chunks - 1
        wait_gather(jt, jt % 4)
        wait_store(jt - 2, (jt - 2) % 4)
        add_pe(jt % 4, jt % 2)
        fire_store(jt, jt % 4)

        wait_store(n_chunks - 2, (n_chunks - 2) % 4)
        wait_store(n_chunks - 1, (n_chunks - 1) % 4)

    return body


def kernel(sequence, token_table):
    batch, seq_len = sequence.shape
    vocab, d_model = token_table.shape
    n_rows = batch * seq_len
    pe = _pos_table(seq_len, d_model)
    seq32 = sequence.astype(jnp.int32)
    fidx = ((seq32 & jnp.int32(~511)) | ((seq32 & 127) << 2)
            | ((seq32 >> 7) & 3))
    idx = fidx.reshape(NW, n_rows // NW // CHUNK, CHUNK)

    tabT = token_table.T
    tab_pairs = _phase_a(tabT)
    n_flat = tab_pairs.shape[0] * 4
    tab_flat = tab_pairs.reshape(n_flat, d_model // 2)
    out = _phase_b(n_rows, n_flat)(tab_flat, idx, pe)
    return out[:, :d_model].reshape(batch, seq_len, d_model)


# speedup vs baseline: 2.0514x; 2.0514x over previous
import functools

import jax
import jax.numpy as jnp
import numpy as np
from jax import lax
from jax.experimental import pallas as pl
from jax.experimental.pallas import tpu as pltpu
from jax.experimental.pallas import tpu_sc as plsc

NC = 2
NS = 16
NW = NC * NS

D = 64
PERIOD = 200
CHUNK = 100
VB = 2048
NBLK = 489


def _pos_table(seq_len, d_model):
    pos = jnp.arange(seq_len, dtype=jnp.float32)[:, None]
    div = jnp.exp(jnp.arange(0, d_model, 2, dtype=jnp.float32)
                  * -(np.log(10000.0) / d_model))
    pe = jnp.zeros((seq_len, d_model), dtype=jnp.float32)
    pe = pe.at[:, 0::2].set(jnp.sin(pos * div))
    pe = pe.at[:, 1::2].set(jnp.cos(pos * div))
    return pe


def _phase_a(tabT):
    def body(in_ref, out_ref):
        t = in_ref[...].T
        u = jax.lax.bitcast_convert_type(t, jnp.int32)
        lo = lax.shift_right_logical(u[:, : D // 2], 16)
        hi = u[:, D // 2:] & jnp.int32(~0xFFFF)
        w = hi | lo
        q = VB // 4
        out_ref[...] = jnp.concatenate(
            [w[:q], w[q:2 * q], w[2 * q:3 * q], w[3 * q:]], axis=1)

    return pl.pallas_call(
        body,
        grid=(NBLK,),
        in_specs=[pl.BlockSpec((D, VB), lambda i: (0, i))],
        out_specs=pl.BlockSpec((VB // 4, 128), lambda i: (i, 0)),
        out_shape=jax.ShapeDtypeStruct((NBLK * VB // 4, 128), jnp.int32),
    )(tabT)


def _phase_b(n_rows, n_flat):
    per_w = n_rows // NW
    n_chunks = per_w // CHUNK
    mesh = plsc.VectorSubcoreMesh(core_axis_name="c", subcore_axis_name="s")

    @functools.partial(
        pl.kernel,
        mesh=mesh,
        compiler_params=pltpu.CompilerParams(
            use_tc_tiling_on_sc=False, needs_layout_passes=False),
        out_type=jax.ShapeDtypeStruct((n_rows, 128), jnp.float32),
        scratch_types=[
            pltpu.VMEM((n_chunks, CHUNK), jnp.int32),
            pltpu.VMEM((PERIOD, D), jnp.float32),
            pltpu.VMEM((4, CHUNK, D // 2), jnp.int32),
            pltpu.VMEM((4, CHUNK, D), jnp.float32),
            pltpu.SemaphoreType.DMA((4,)),
            pltpu.SemaphoreType.DMA((4,)),
        ],
    )
    def body(tab_hbm, idx_hbm, pe_hbm, out_hbm, idx_v, pe_v, gbuf, sbuf,
             gsem, ssem):
        wid = lax.axis_index("s") * NC + lax.axis_index("c")
        rbase = wid * per_w

        pltpu.sync_copy(idx_hbm.at[wid], idx_v)
        pltpu.sync_copy(pe_hbm, pe_v)

        def fire_gather(j, b):
            pltpu.async_copy(tab_hbm.at[idx_v.at[j]], gbuf.at[b], gsem.at[b])

        def wait_gather(j, b):
            pltpu.make_async_copy(
                tab_hbm.at[idx_v.at[j]], gbuf.at[b], gsem.at[b]).wait()

        def fire_store(j, b):
            pltpu.async_copy(sbuf.at[b],
                             out_hbm.at[pl.ds(rbase + j * CHUNK, CHUNK),
                                        pl.ds(0, D)],
                             ssem.at[b])

        def wait_store(j, b):
            pltpu.make_async_copy(
                sbuf.at[b],
                out_hbm.at[pl.ds(rbase + j * CHUNK, CHUNK), pl.ds(0, D)],
                ssem.at[b]).wait()

        def add_pe(b, parity):
            pbase = parity * CHUNK

            def row_body(r, carry):
                for g in range(D // 32):
                    x32 = gbuf[b, r, pl.ds(g * 16, 16)]
                    x = plsc.bitcast(x32, jnp.bfloat16)
                    lo, hi = plsc.unpack(x, format=plsc.PackFormat.INTERLEAVED)
                    c0, c1 = g * 16, D // 2 + g * 16
                    sbuf[b, r, pl.ds(c0, 16)] = lo + pe_v[pbase + r, pl.ds(c0, 16)]
                    sbuf[b, r, pl.ds(c1, 16)] = hi + pe_v[pbase + r, pl.ds(c1, 16)]
                return carry

            lax.fori_loop(0, CHUNK, row_body, 0, unroll=2)

        fire_gather(0, 0)
        fire_gather(1, 1)

        wait_gather(0, 0)
        add_pe(0, 0)
        fire_store(0, 0)
        fire_gather(2, 2)

        wait_gather(1, 1)
        add_pe(1, 1)
        fire_store(1, 1)
        fire_gather(3, 3)

        def steady(jj, carry):
            j0 = 2 + jj * 4
            for b_off in range(4):
                j = j0 + b_off
                b = (2 + b_off) % 4
                parity = b_off % 2
                wait_gather(j, b)
                wait_store(j - 2, (b + 2) % 4)
                add_pe(b, parity)
                fire_store(j, b)
                fire_gather(j + 2, (b + 2) % 4)
            return carry

        lax.fori_loop(0, (n_chunks - 4) // 4, steady, 0)

        jt = n_chunks - 2
        wait_gather(jt, jt % 4)
        wait_store(jt - 2, (jt - 2) % 4)
        add_pe(jt % 4, jt % 2)
        fire_store(jt, jt % 4)

        jt = n_chunks - 1
        wait_gather(jt, jt % 4)
        wait_store(jt - 2, (jt - 2) % 4)
        add_pe(jt % 4, jt % 2)
        fire_store(jt, jt % 4)

        wait_store(n_chunks - 2, (n_chunks - 2) % 4)
        wait_store(n_chunks - 1, (n_chunks - 1) % 4)

    return body


def kernel(sequence, token_table):
    batch, seq_len = sequence.shape
    vocab, d_model = token_table.shape
    n_rows = batch * seq_len
    pe = _pos_table(seq_len, d_model)
    seq32 = sequence.astype(jnp.int32)
    fidx = ((seq32 & jnp.int32(~(VB - 1))) | ((seq32 & (VB // 4 - 1)) << 2)
            | ((seq32 >> (VB.bit_length() - 3)) & 3))
    idx = fidx.reshape(NW, n_rows // NW // CHUNK, CHUNK)

    tabT = token_table.T
    tab_pairs = _phase_a(tabT)
    n_flat = tab_pairs.shape[0] * 4
    tab_flat = tab_pairs.reshape(n_flat, d_model // 2)
    out = _phase_b(n_rows, n_flat)(tab_flat, idx, pe)
    return out[:, :d_model].reshape(batch, seq_len, d_model)


# speedup vs baseline: 2.4832x; 1.2105x over previous
import functools

import jax
import jax.numpy as jnp
import numpy as np
from jax import lax
from jax.experimental import pallas as pl
from jax.experimental.pallas import tpu as pltpu
from jax.experimental.pallas import tpu_sc as plsc

NC = 2
NS = 16
NW = NC * NS

D = 64
PERIOD = 200
CHUNK = 100
VB = 2048
NBLK = 489


def _pos_table(seq_len, d_model):
    pos = jnp.arange(seq_len, dtype=jnp.float32)[:, None]
    div = jnp.exp(jnp.arange(0, d_model, 2, dtype=jnp.float32)
                  * -(np.log(10000.0) / d_model))
    pe = jnp.zeros((seq_len, d_model), dtype=jnp.float32)
    pe = pe.at[:, 0::2].set(jnp.sin(pos * div))
    pe = pe.at[:, 1::2].set(jnp.cos(pos * div))
    return pe


def _phase_a(tabT):
    def body(in_ref, out_ref):
        t = in_ref[...].T
        h = VB // 2
        out_ref[...] = jnp.concatenate([t[:h], t[h:]], axis=1)

    return pl.pallas_call(
        body,
        grid=(NBLK,),
        in_specs=[pl.BlockSpec((D, VB), lambda i: (0, i))],
        out_specs=pl.BlockSpec((VB // 2, 128), lambda i: (i, 0)),
        out_shape=jax.ShapeDtypeStruct((NBLK * VB // 2, 128), jnp.float32),
    )(tabT)


def _phase_b(n_rows, n_flat):
    per_w = n_rows // NW
    n_chunks = per_w // CHUNK
    mesh = plsc.VectorSubcoreMesh(core_axis_name="c", subcore_axis_name="s")

    @functools.partial(
        pl.kernel,
        mesh=mesh,
        compiler_params=pltpu.CompilerParams(
            use_tc_tiling_on_sc=False, needs_layout_passes=False),
        out_type=jax.ShapeDtypeStruct((n_rows, 128), jnp.float32),
        scratch_types=[
            pltpu.VMEM((n_chunks, CHUNK), jnp.int32),
            pltpu.VMEM((PERIOD, D), jnp.float32),
            pltpu.VMEM((4, CHUNK, D), jnp.float32),
            pltpu.SemaphoreType.DMA((4,)),
            pltpu.SemaphoreType.DMA((4,)),
        ],
    )
    def body(tab_hbm, idx_hbm, pe_hbm, out_hbm, idx_v, pe_v, gbuf,
             gsem, ssem):
        wid = lax.axis_index("s") * NC + lax.axis_index("c")
        rbase = wid * per_w

        pltpu.sync_copy(idx_hbm.at[wid], idx_v)
        pltpu.sync_copy(pe_hbm, pe_v)

        def fire_gather(j, b):
            pltpu.async_copy(tab_hbm.at[idx_v.at[j]], gbuf.at[b], gsem.at[b])

        def wait_gather(j, b):
            pltpu.make_async_copy(
                tab_hbm.at[idx_v.at[j]], gbuf.at[b], gsem.at[b]).wait()

        def fire_store(j, b):
            pltpu.async_copy(gbuf.at[b],
                             out_hbm.at[pl.ds(rbase + j * CHUNK, CHUNK),
                                        pl.ds(0, D)],
                             ssem.at[b])

        def wait_store(j, b):
            pltpu.make_async_copy(
                gbuf.at[b],
                out_hbm.at[pl.ds(rbase + j * CHUNK, CHUNK), pl.ds(0, D)],
                ssem.at[b]).wait()

        def add_pe(b, parity):
            pbase = parity * CHUNK

            def row_body(r, carry):
                for c in range(D // 16):
                    vec = pe_v[pbase + r, pl.ds(c * 16, 16)]
                    plsc.addupdate(gbuf.at[b, r, pl.ds(c * 16, 16)], vec)
                return carry

            lax.fori_loop(0, CHUNK, row_body, 0, unroll=4)

        fire_gather(0, 0)
        fire_gather(1, 1)

        wait_gather(0, 0)
        add_pe(0, 0)
        fire_store(0, 0)
        fire_gather(2, 2)

        wait_gather(1, 1)
        add_pe(1, 1)
        fire_store(1, 1)
        fire_gather(3, 3)

        def steady(jj, carry):
            j0 = 2 + jj * 4
            for b_off in range(4):
                j = j0 + b_off
                b = (2 + b_off) % 4
                parity = b_off % 2
                wait_gather(j, b)
                wait_store(j - 2, (b + 2) % 4)
                add_pe(b, parity)
                fire_store(j, b)
                fire_gather(j + 2, (b + 2) % 4)
            return carry

        lax.fori_loop(0, (n_chunks - 4) // 4, steady, 0)

        jt = n_chunks - 2
        wait_gather(jt, jt % 4)
        wait_store(jt - 2, (jt - 2) % 4)
        add_pe(jt % 4, jt % 2)
        fire_store(jt, jt % 4)

        jt = n_chunks - 1
        wait_gather(jt, jt % 4)
        wait_store(jt - 2, (jt - 2) % 4)
        add_pe(jt % 4, jt % 2)
        fire_store(jt, jt % 4)

        wait_store(n_chunks - 2, (n_chunks - 2) % 4)
        wait_store(n_chunks - 1, (n_chunks - 1) % 4)

    return body


def kernel(sequence, token_table):
    batch, seq_len = sequence.shape
    vocab, d_model = token_table.shape
    n_rows = batch * seq_len
    pe = _pos_table(seq_len, d_model)
    seq32 = sequence.astype(jnp.int32)
    fidx = ((seq32 & jnp.int32(~(VB - 1))) | ((seq32 & (VB // 2 - 1)) << 1)
            | ((seq32 >> (VB.bit_length() - 2)) & 1))
    idx = fidx.reshape(NW, n_rows // NW // CHUNK, CHUNK)

    tabT = token_table.T
    tab_pairs = _phase_a(tabT)
    n_flat = tab_pairs.shape[0] * 2
    tab_flat = tab_pairs.reshape(n_flat, d_model)
    out = _phase_b(n_rows, n_flat)(tab_flat, idx, pe)
    return out[:, :d_model].reshape(batch, seq_len, d_model)


# speedup vs baseline: 3.7146x; 1.4959x over previous
import functools

import jax
import jax.numpy as jnp
import numpy as np
from jax import lax
from jax.experimental import pallas as pl
from jax.experimental.pallas import tpu as pltpu
from jax.experimental.pallas import tpu_sc as plsc

NC = 2
NS = 16
NW = NC * NS

D = 64
PERIOD = 200
CHUNK = 100
VB = 8192
NBLK = 123


def _pos_table(seq_len, d_model):
    pos = jnp.arange(seq_len, dtype=jnp.float32)[:, None]
    div = jnp.exp(jnp.arange(0, d_model, 2, dtype=jnp.float32)
                  * -(np.log(10000.0) / d_model))
    pe = jnp.zeros((seq_len, d_model), dtype=jnp.float32)
    pe = pe.at[:, 0::2].set(jnp.sin(pos * div))
    pe = pe.at[:, 1::2].set(jnp.cos(pos * div))
    return pe


def _phase_a(tabT):
    def body(in_ref, out_ref):
        t = in_ref[...].T
        h = VB // 2
        out_ref[...] = jnp.concatenate([t[:h], t[h:]], axis=1)

    return pl.pallas_call(
        body,
        grid=(NBLK,),
        in_specs=[pl.BlockSpec((D, VB), lambda i: (0, i))],
        out_specs=pl.BlockSpec((VB // 2, 128), lambda i: (i, 0)),
        out_shape=jax.ShapeDtypeStruct((NBLK * VB // 2, 128), jnp.float32),
    )(tabT)


def _phase_b(n_rows, n_flat):
    per_w = n_rows // NW
    n_chunks = per_w // CHUNK
    mesh = plsc.VectorSubcoreMesh(core_axis_name="c", subcore_axis_name="s")

    @functools.partial(
        pl.kernel,
        mesh=mesh,
        compiler_params=pltpu.CompilerParams(
            use_tc_tiling_on_sc=False, needs_layout_passes=False),
        out_type=jax.ShapeDtypeStruct((n_rows, 128), jnp.float32),
        scratch_types=[
            pltpu.VMEM((n_chunks, CHUNK), jnp.int32),
            pltpu.VMEM((PERIOD, D), jnp.float32),
            pltpu.VMEM((4, CHUNK, D), jnp.float32),
            pltpu.SemaphoreType.DMA((4,)),
            pltpu.SemaphoreType.DMA((4,)),
        ],
    )
    def body(tab_hbm, idx_hbm, pe_hbm, out_hbm, idx_v, pe_v, gbuf,
             gsem, ssem):
        wid = lax.axis_index("s") * NC + lax.axis_index("c")
        rbase = wid * per_w

        pltpu.sync_copy(idx_hbm.at[wid], idx_v)
        pltpu.sync_copy(pe_hbm, pe_v)

        def fire_gather(j, b):
            pltpu.async_copy(tab_hbm.at[idx_v.at[j]], gbuf.at[b], gsem.at[b])

        def wait_gather(j, b):
            pltpu.make_async_copy(
                tab_hbm.at[idx_v.at[j]], gbuf.at[b], gsem.at[b]).wait()

        def fire_store(j, b):
            pltpu.async_copy(gbuf.at[b],
                             out_hbm.at[pl.ds(rbase + j * CHUNK, CHUNK),
                                        pl.ds(0, D)],
                             ssem.at[b])

        def wait_store(j, b):
            pltpu.make_async_copy(
                gbuf.at[b],
                out_hbm.at[pl.ds(rbase + j * CHUNK, CHUNK), pl.ds(0, D)],
                ssem.at[b]).wait()

        def add_pe(b, parity):
            pbase = parity * CHUNK

            def row_body(r, carry):
                for c in range(D // 16):
                    vec = pe_v[pbase + r, pl.ds(c * 16, 16)]
                    plsc.addupdate(gbuf.at[b, r, pl.ds(c * 16, 16)], vec)
                return carry

            lax.fori_loop(0, CHUNK, row_body, 0, unroll=4)

        fire_gather(0, 0)
        fire_gather(1, 1)

        wait_gather(0, 0)
        add_pe(0, 0)
        fire_store(0, 0)
        fire_gather(2, 2)

        wait_gather(1, 1)
        add_pe(1, 1)
        fire_store(1, 1)
        fire_gather(3, 3)

        def steady(jj, carry):
            j0 = 2 + jj * 4
            for b_off in range(4):
                j = j0 + b_off
                b = (2 + b_off) % 4
                parity = b_off % 2
                wait_gather(j, b)
                wait_store(j - 2, (b + 2) % 4)
                add_pe(b, parity)
                fire_store(j, b)
                fire_gather(j + 2, (b + 2) % 4)
            return carry

        lax.fori_loop(0, (n_chunks - 4) // 4, steady, 0)

        jt = n_chunks - 2
        wait_gather(jt, jt % 4)
        wait_store(jt - 2, (jt - 2) % 4)
        add_pe(jt % 4, jt % 2)
        fire_store(jt, jt % 4)

        jt = n_chunks - 1
        wait_gather(jt, jt % 4)
        wait_store(jt - 2, (jt - 2) % 4)
        add_pe(jt % 4, jt % 2)
        fire_store(jt, jt % 4)

        wait_store(n_chunks - 2, (n_chunks - 2) % 4)
        wait_store(n_chunks - 1, (n_chunks - 1) % 4)

    return body


def kernel(sequence, token_table):
    batch, seq_len = sequence.shape
    vocab, d_model = token_table.shape
    n_rows = batch * seq_len
    pe = _pos_table(seq_len, d_model)
    seq32 = sequence.astype(jnp.int32)
    fidx = ((seq32 & jnp.int32(~(VB - 1))) | ((seq32 & (VB // 2 - 1)) << 1)
            | ((seq32 >> (VB.bit_length() - 2)) & 1))
    idx = fidx.reshape(NW, n_rows // NW // CHUNK, CHUNK)

    tabT = token_table.T
    tab_pairs = _phase_a(tabT)
    n_flat = tab_pairs.shape[0] * 2
    tab_flat = tab_pairs.reshape(n_flat, d_model)
    out = _phase_b(n_rows, n_flat)(tab_flat, idx, pe)
    return out[:, :d_model].reshape(batch, seq_len, d_model)


# speedup vs baseline: 4.0641x; 1.0941x over previous
import functools

import jax
import jax.numpy as jnp
import numpy as np
from jax import lax
from jax.experimental import pallas as pl
from jax.experimental.pallas import tpu as pltpu
from jax.experimental.pallas import tpu_sc as plsc

NC = 2
NS = 16
NW = NC * NS

D = 64
PERIOD = 200
CHUNK = 100
VB = 16384
NBLK = 62


def _pos_table(seq_len, d_model):
    pos = jnp.arange(seq_len, dtype=jnp.float32)[:, None]
    div = jnp.exp(jnp.arange(0, d_model, 2, dtype=jnp.float32)
                  * -(np.log(10000.0) / d_model))
    pe = jnp.zeros((seq_len, d_model), dtype=jnp.float32)
    pe = pe.at[:, 0::2].set(jnp.sin(pos * div))
    pe = pe.at[:, 1::2].set(jnp.cos(pos * div))
    return pe


def _phase_a(tabT):
    def body(in_ref, out_ref):
        t = in_ref[...].T
        h = VB // 2
        out_ref[...] = jnp.concatenate([t[:h], t[h:]], axis=1)

    return pl.pallas_call(
        body,
        grid=(NBLK,),
        in_specs=[pl.BlockSpec((D, VB), lambda i: (0, i))],
        out_specs=pl.BlockSpec((VB // 2, 128), lambda i: (i, 0)),
        out_shape=jax.ShapeDtypeStruct((NBLK * VB // 2, 128), jnp.float32),
    )(tabT)


def _phase_b(n_rows, n_flat):
    per_w = n_rows // NW
    n_chunks = per_w // CHUNK
    mesh = plsc.VectorSubcoreMesh(core_axis_name="c", subcore_axis_name="s")

    @functools.partial(
        pl.kernel,
        mesh=mesh,
        compiler_params=pltpu.CompilerParams(
            use_tc_tiling_on_sc=False, needs_layout_passes=False),
        out_type=jax.ShapeDtypeStruct((n_rows, 128), jnp.float32),
        scratch_types=[
            pltpu.VMEM((n_chunks, CHUNK), jnp.int32),
            pltpu.VMEM((PERIOD, D), jnp.float32),
            pltpu.VMEM((4, CHUNK, D), jnp.float32),
            pltpu.SemaphoreType.DMA((4,)),
            pltpu.SemaphoreType.DMA((4,)),
        ],
    )
    def body(tab_hbm, idx_hbm, pe_hbm, out_hbm, idx_v, pe_v, gbuf,
             gsem, ssem):
        wid = lax.axis_index("s") * NC + lax.axis_index("c")
        rbase = wid * per_w

        pltpu.sync_copy(idx_hbm.at[wid], idx_v)
        pltpu.sync_copy(pe_hbm, pe_v)

        def fire_gather(j, b):
            pltpu.async_copy(tab_hbm.at[idx_v.at[j]], gbuf.at[b], gsem.at[b])

        def wait_gather(j, b):
            pltpu.make_async_copy(
                tab_hbm.at[idx_v.at[j]], gbuf.at[b], gsem.at[b]).wait()

        def fire_store(j, b):
            pltpu.async_copy(gbuf.at[b],
                             out_hbm.at[pl.ds(rbase + j * CHUNK, CHUNK),
                                        pl.ds(0, D)],
                             ssem.at[b])

        def wait_store(j, b):
            pltpu.make_async_copy(
                gbuf.at[b],
                out_hbm.at[pl.ds(rbase + j * CHUNK, CHUNK), pl.ds(0, D)],
                ssem.at[b]).wait()

        def add_pe(b, parity):
            pbase = parity * CHUNK

            def row_body(r, carry):
                for c in range(D // 16):
                    vec = pe_v[pbase + r, pl.ds(c * 16, 16)]
                    plsc.addupdate(gbuf.at[b, r, pl.ds(c * 16, 16)], vec)
                return carry

            lax.fori_loop(0, CHUNK, row_body, 0, unroll=4)

        fire_gather(0, 0)
        fire_gather(1, 1)

        wait_gather(0, 0)
        add_pe(0, 0)
        fire_store(0, 0)
        fire_gather(2, 2)

        wait_gather(1, 1)
        add_pe(1, 1)
        fire_store(1, 1)
        fire_gather(3, 3)

        def steady(jj, carry):
            j0 = 2 + jj * 4
            for b_off in range(4):
                j = j0 + b_off
                b = (2 + b_off) % 4
                parity = b_off % 2
                wait_gather(j, b)
                wait_store(j - 2, (b + 2) % 4)
                add_pe(b, parity)
                fire_store(j, b)
                fire_gather(j + 2, (b + 2) % 4)
            return carry

        lax.fori_loop(0, (n_chunks - 4) // 4, steady, 0)

        jt = n_chunks - 2
        wait_gather(jt, jt % 4)
        wait_store(jt - 2, (jt - 2) % 4)
        add_pe(jt % 4, jt % 2)
        fire_store(jt, jt % 4)

        jt = n_chunks - 1
        wait_gather(jt, jt % 4)
        wait_store(jt - 2, (jt - 2) % 4)
        add_pe(jt % 4, jt % 2)
        fire_store(jt, jt % 4)

        wait_store(n_chunks - 2, (n_chunks - 2) % 4)
        wait_store(n_chunks - 1, (n_chunks - 1) % 4)

    return body


def kernel(sequence, token_table):
    batch, seq_len = sequence.shape
    vocab, d_model = token_table.shape
    n_rows = batch * seq_len
    pe = _pos_table(seq_len, d_model)
    seq32 = sequence.astype(jnp.int32)
    fidx = ((seq32 & jnp.int32(~(VB - 1))) | ((seq32 & (VB // 2 - 1)) << 1)
            | ((seq32 >> (VB.bit_length() - 2)) & 1))
    idx = fidx.reshape(NW, n_rows // NW // CHUNK, CHUNK)

    tabT = token_table.T
    tab_pairs = _phase_a(tabT)
    n_flat = tab_pairs.shape[0] * 2
    tab_flat = tab_pairs.reshape(n_flat, d_model)
    out = _phase_b(n_rows, n_flat)(tab_flat, idx, pe)
    return out[:, :d_model].reshape(batch, seq_len, d_model)


# speedup vs baseline: 4.2494x; 1.0456x over previous
import functools

import jax
import jax.numpy as jnp
import numpy as np
from jax import lax
from jax.experimental import pallas as pl
from jax.experimental.pallas import tpu as pltpu
from jax.experimental.pallas import tpu_sc as plsc

NC = 2
NS = 16
NW = NC * NS

D = 64
PERIOD = 200
CHUNK = 100
VB = 32768
NBLK = 31


def _pos_table(seq_len, d_model):
    pos = jnp.arange(seq_len, dtype=jnp.float32)[:, None]
    div = jnp.exp(jnp.arange(0, d_model, 2, dtype=jnp.float32)
                  * -(np.log(10000.0) / d_model))
    pe = jnp.zeros((seq_len, d_model), dtype=jnp.float32)
    pe = pe.at[:, 0::2].set(jnp.sin(pos * div))
    pe = pe.at[:, 1::2].set(jnp.cos(pos * div))
    return pe


def _phase_a(tabT):
    def body(in_ref, out_ref):
        t = in_ref[...].T
        h = VB // 2
        out_ref[...] = jnp.concatenate([t[:h], t[h:]], axis=1)

    return pl.pallas_call(
        body,
        grid=(NBLK,),
        in_specs=[pl.BlockSpec((D, VB), lambda i: (0, i))],
        out_specs=pl.BlockSpec((VB // 2, 128), lambda i: (i, 0)),
        out_shape=jax.ShapeDtypeStruct((NBLK * VB // 2, 128), jnp.float32),
    )(tabT)


def _phase_b(n_rows, n_flat):
    per_w = n_rows // NW
    n_chunks = per_w // CHUNK
    mesh = plsc.VectorSubcoreMesh(core_axis_name="c", subcore_axis_name="s")

    @functools.partial(
        pl.kernel,
        mesh=mesh,
        compiler_params=pltpu.CompilerParams(
            use_tc_tiling_on_sc=False, needs_layout_passes=False),
        out_type=jax.ShapeDtypeStruct((n_rows, 128), jnp.float32),
        scratch_types=[
            pltpu.VMEM((n_chunks, CHUNK), jnp.int32),
            pltpu.VMEM((PERIOD, D), jnp.float32),
            pltpu.VMEM((4, CHUNK, D), jnp.float32),
            pltpu.SemaphoreType.DMA((4,)),
            pltpu.SemaphoreType.DMA((4,)),
        ],
    )
    def body(tab_hbm, idx_hbm, pe_hbm, out_hbm, idx_v, pe_v, gbuf,
             gsem, ssem):
        wid = lax.axis_index("s") * NC + lax.axis_index("c")
        rbase = wid * per_w

        pltpu.sync_copy(idx_hbm.at[wid], idx_v)
        pltpu.sync_copy(pe_hbm, pe_v)

        def fire_gather(j, b):
            pltpu.async_copy(tab_hbm.at[idx_v.at[j]], gbuf.at[b], gsem.at[b])

        def wait_gather(j, b):
            pltpu.make_async_copy(
                tab_hbm.at[idx_v.at[j]], gbuf.at[b], gsem.at[b]).wait()

        def fire_store(j, b):
            pltpu.async_copy(gbuf.at[b],
                             out_hbm.at[pl.ds(rbase + j * CHUNK, CHUNK),
                                        pl.ds(0, D)],
                             ssem.at[b])

        def wait_store(j, b):
            pltpu.make_async_copy(
                gbuf.at[b],
                out_hbm.at[pl.ds(rbase + j * CHUNK, CHUNK), pl.ds(0, D)],
                ssem.at[b]).wait()

        def add_pe(b, parity):
            pbase = parity * CHUNK

            def row_body(r, carry):
                for c in range(D // 16):
                    vec = pe_v[pbase + r, pl.ds(c * 16, 16)]
                    plsc.addupdate(gbuf.at[b, r, pl.ds(c * 16, 16)], vec)
                return carry

            lax.fori_loop(0, CHUNK, row_body, 0, unroll=4)

        fire_gather(0, 0)
        fire_gather(1, 1)

        wait_gather(0, 0)
        add_pe(0, 0)
        fire_store(0, 0)
        fire_gather(2, 2)

        wait_gather(1, 1)
        add_pe(1, 1)
        fire_store(1, 1)
        fire_gather(3, 3)

        def steady(jj, carry):
            j0 = 2 + jj * 4
            for b_off in range(4):
                j = j0 + b_off
                b = (2 + b_off) % 4
                parity = b_off % 2
                wait_gather(j, b)
                wait_store(j - 2, (b + 2) % 4)
                add_pe(b, parity)
                fire_store(j, b)
                fire_gather(j + 2, (b + 2) % 4)
            return carry

        lax.fori_loop(0, (n_chunks - 4) // 4, steady, 0)

        jt = n_chunks - 2
        wait_gather(jt, jt % 4)
        wait_store(jt - 2, (jt - 2) % 4)
        add_pe(jt % 4, jt % 2)
        fire_store(jt, jt % 4)

        jt = n_chunks - 1
        wait_gather(jt, jt % 4)
        wait_store(jt - 2, (jt - 2) % 4)
        add_pe(jt % 4, jt % 2)
        fire_store(jt, jt % 4)

        wait_store(n_chunks - 2, (n_chunks - 2) % 4)
        wait_store(n_chunks - 1, (n_chunks - 1) % 4)

    return body


def kernel(sequence, token_table):
    batch, seq_len = sequence.shape
    vocab, d_model = token_table.shape
    n_rows = batch * seq_len
    pe = _pos_table(seq_len, d_model)
    seq32 = sequence.astype(jnp.int32)
    fidx = ((seq32 & jnp.int32(~(VB - 1))) | ((seq32 & (VB // 2 - 1)) << 1)
            | ((seq32 >> (VB.bit_length() - 2)) & 1))
    idx = fidx.reshape(NW, n_rows // NW // CHUNK, CHUNK)

    tabT = token_table.T
    tab_pairs = _phase_a(tabT)
    n_flat = tab_pairs.shape[0] * 2
    tab_flat = tab_pairs.reshape(n_flat, d_model)
    out = _phase_b(n_rows, n_flat)(tab_flat, idx, pe)
    return out[:, :d_model].reshape(batch, seq_len, d_model)
